# initial kernel scaffold (unmeasured)
import jax
import jax.numpy as jnp
from jax import lax
from jax.experimental import pallas as pl
from jax.experimental.pallas import tpu as pltpu

N_EXP = 8
TOPK = 2


def _ag_xr_body(x_ref, r_ref, ox_ref, or_ref, send_sems, recv_sems):
    px = lax.axis_index("x")
    py = lax.axis_index("y")
    peer = (1 - px, py)

    ox_ref[pl.ds(px, 1)] = x_ref[...][None]
    or_ref[pl.ds(px, 1)] = r_ref[...][None]

    rdma_x = pltpu.make_async_remote_copy(
        src_ref=x_ref,
        dst_ref=ox_ref.at[px],
        send_sem=send_sems.at[0],
        recv_sem=recv_sems.at[0],
        device_id=peer,
        device_id_type=pl.DeviceIdType.MESH,
    )
    rdma_r = pltpu.make_async_remote_copy(
        src_ref=r_ref,
        dst_ref=or_ref.at[px],
        send_sem=send_sems.at[1],
        recv_sem=recv_sems.at[1],
        device_id=peer,
        device_id_type=pl.DeviceIdType.MESH,
    )
    rdma_x.start()
    rdma_r.start()
    rdma_x.wait()
    rdma_r.wait()


def _ag_xr(x_bf, r_shard):
    t_half, d = x_bf.shape
    t, e_loc = r_shard.shape
    return pl.pallas_call(
        _ag_xr_body,
        out_shape=(
            jax.ShapeDtypeStruct((2, t_half, d), jnp.bfloat16),
            jax.ShapeDtypeStruct((2, t, e_loc), jnp.float32),
        ),
        in_specs=[
            pl.BlockSpec(memory_space=pltpu.VMEM),
            pl.BlockSpec(memory_space=pltpu.VMEM),
        ],
        out_specs=(
            pl.BlockSpec(memory_space=pltpu.VMEM),
            pl.BlockSpec(memory_space=pltpu.VMEM),
        ),
        scratch_shapes=[
            pltpu.SemaphoreType.DMA((2,)),
            pltpu.SemaphoreType.DMA((2,)),
        ],
        compiler_params=pltpu.CompilerParams(collective_id=0),
    )(x_bf, r_shard)


def _ag_w_body(w_ref, ow_ref, send_sem, recv_sem):
    px = lax.axis_index("x")
    py = lax.axis_index("y")
    ow_ref[pl.ds(px, 1)] = w_ref[...][None]
    rdma = pltpu.make_async_remote_copy(
        src_ref=w_ref,
        dst_ref=ow_ref.at[px],
        send_sem=send_sem,
        recv_sem=recv_sem,
        device_id=(1 - px, py),
        device_id_type=pl.DeviceIdType.MESH,
    )
    rdma.start()
    rdma.wait()


def _ag_w(w_own):
    t_half, ne = w_own.shape
    return pl.pallas_call(
        _ag_w_body,
        out_shape=jax.ShapeDtypeStruct((2, t_half, ne), jnp.float32),
        in_specs=[pl.BlockSpec(memory_space=pltpu.VMEM)],
        out_specs=pl.BlockSpec(memory_space=pltpu.VMEM),
        scratch_shapes=[
            pltpu.SemaphoreType.DMA(()),
            pltpu.SemaphoreType.DMA(()),
        ],
        compiler_params=pltpu.CompilerParams(collective_id=1),
    )(w_own)


def _moe_body(x_ref, w1_ref, w2_ref, wg_ref, out_ref):
    e = pl.program_id(0)
    h = jnp.dot(
        x_ref[...],
        w1_ref[0].astype(jnp.bfloat16),
        preferred_element_type=jnp.float32,
    )
    h = jnp.maximum(h, 0.0).astype(jnp.bfloat16)
    contrib = jnp.dot(
        h, w2_ref[0].astype(jnp.bfloat16), preferred_element_type=jnp.float32
    )
    contrib = contrib * wg_ref[...]

    @pl.when(e == 0)
    def _():
        out_ref[...] = contrib

    @pl.when(e != 0)
    def _():
        out_ref[...] += contrib


def _moe_compute(x_full, W1, W2, w_loc):
    t, d = x_full.shape
    e_loc, _, f = W1.shape
    return pl.pallas_call(
        _moe_body,
        grid=(e_loc,),
        in_specs=[
            pl.BlockSpec((t, d), lambda e: (0, 0)),
            pl.BlockSpec((1, d, f), lambda e: (e, 0, 0)),
            pl.BlockSpec((1, f, d), lambda e: (e, 0, 0)),
            pl.BlockSpec((t, 1), lambda e: (0, e)),
        ],
        out_specs=pl.BlockSpec((t, d), lambda e: (0, 0)),
        out_shape=jax.ShapeDtypeStruct((t, d), jnp.float32),
        compiler_params=pltpu.CompilerParams(
            dimension_semantics=("arbitrary",)
        ),
    )(x_full, W1, W2, w_loc)


def _combine_body(p_ref, out_ref, send_buf, recv_buf, send_sem, recv_sem):
    px = lax.axis_index("x")
    py = lax.axis_index("y")
    peer_x = 1 - px
    t_half = out_ref.shape[0]

    send_buf[...] = p_ref[pl.ds(peer_x * t_half, t_half), :].astype(
        jnp.bfloat16
    )
    rdma = pltpu.make_async_remote_copy(
        src_ref=send_buf,
        dst_ref=recv_buf,
        send_sem=send_sem,
        recv_sem=recv_sem,
        device_id=(peer_x, py),
        device_id_type=pl.DeviceIdType.MESH,
    )
    rdma.start()
    rdma.wait()
    out_ref[...] = p_ref[pl.ds(px * t_half, t_half), :] + recv_buf[
        ...
    ].astype(jnp.float32)


def _combine(partial, t_half):
    t, d = partial.shape
    return pl.pallas_call(
        _combine_body,
        out_shape=jax.ShapeDtypeStruct((t_half, d), jnp.float32),
        in_specs=[pl.BlockSpec(memory_space=pltpu.VMEM)],
        out_specs=pl.BlockSpec(memory_space=pltpu.VMEM),
        scratch_shapes=[
            pltpu.VMEM((t_half, d), jnp.bfloat16),
            pltpu.VMEM((t_half, d), jnp.bfloat16),
            pltpu.SemaphoreType.DMA(()),
            pltpu.SemaphoreType.DMA(()),
        ],
        compiler_params=pltpu.CompilerParams(collective_id=2),
    )(partial)


def kernel(x, router, W1, W2):
    t_half, d = x.shape
    e_loc = W1.shape[0]
    px = lax.axis_index("x")

    xg, rg = _ag_xr(x.astype(jnp.bfloat16), router)
    x_full = xg.reshape(2 * t_half, d)
    router_full = jnp.concatenate([rg[0], rg[1]], axis=1)

    gates = jnp.dot(x, router_full, precision=lax.Precision.HIGHEST)
    top_v, top_i = lax.top_k(gates, TOPK)
    wts = jax.nn.softmax(top_v, axis=-1)
    w_own = (
        jnp.zeros((t_half, N_EXP), jnp.float32)
        .at[jnp.arange(t_half)[:, None], top_i]
        .add(wts)
    )

    wg = _ag_w(w_own)
    w_full = wg.reshape(2 * t_half, N_EXP)
    w_loc = lax.dynamic_slice_in_dim(w_full, px * e_loc, e_loc, axis=1)

    partial = _moe_compute(x_full, W1, W2, w_loc)

    return _combine(partial, t_half)


# baseline (device time: 312383 ns/iter reference)
import jax
import jax.numpy as jnp
from jax import lax
from jax.experimental import pallas as pl
from jax.experimental.pallas import tpu as pltpu

N_EXP = 8
TOPK = 2


def _ag_xr_body(x_ref, r_ref, ox_ref, or_ref, send_sems, recv_sems):
    px = lax.axis_index("x")
    py = lax.axis_index("y")
    peer = (1 - px, py)

    ox_ref[pl.ds(px, 1)] = x_ref[...][None]
    or_ref[pl.ds(px, 1)] = r_ref[...][None]

    rdma_x = pltpu.make_async_remote_copy(
        src_ref=x_ref,
        dst_ref=ox_ref.at[px],
        send_sem=send_sems.at[0],
        recv_sem=recv_sems.at[0],
        device_id=peer,
        device_id_type=pl.DeviceIdType.MESH,
    )
    rdma_r = pltpu.make_async_remote_copy(
        src_ref=r_ref,
        dst_ref=or_ref.at[px],
        send_sem=send_sems.at[1],
        recv_sem=recv_sems.at[1],
        device_id=peer,
        device_id_type=pl.DeviceIdType.MESH,
    )
    rdma_x.start()
    rdma_r.start()
    rdma_x.wait()
    rdma_r.wait()


def _ag_xr(x_bf, r_shard):
    t_half, d = x_bf.shape
    t, e_loc = r_shard.shape
    return pl.pallas_call(
        _ag_xr_body,
        out_shape=(
            jax.ShapeDtypeStruct((2, t_half, d), jnp.bfloat16),
            jax.ShapeDtypeStruct((2, t, e_loc), jnp.float32),
        ),
        in_specs=[
            pl.BlockSpec(memory_space=pltpu.VMEM),
            pl.BlockSpec(memory_space=pltpu.VMEM),
        ],
        out_specs=(
            pl.BlockSpec(memory_space=pltpu.VMEM),
            pl.BlockSpec(memory_space=pltpu.VMEM),
        ),
        scratch_shapes=[
            pltpu.SemaphoreType.DMA((2,)),
            pltpu.SemaphoreType.DMA((2,)),
        ],
    )(x_bf, r_shard)


def _ag_w_body(w_ref, ow_ref, send_sem, recv_sem):
    px = lax.axis_index("x")
    py = lax.axis_index("y")
    ow_ref[pl.ds(px, 1)] = w_ref[...][None]
    rdma = pltpu.make_async_remote_copy(
        src_ref=w_ref,
        dst_ref=ow_ref.at[px],
        send_sem=send_sem,
        recv_sem=recv_sem,
        device_id=(1 - px, py),
        device_id_type=pl.DeviceIdType.MESH,
    )
    rdma.start()
    rdma.wait()


def _ag_w(w_own):
    t_half, ne = w_own.shape
    return pl.pallas_call(
        _ag_w_body,
        out_shape=jax.ShapeDtypeStruct((2, t_half, ne), jnp.float32),
        in_specs=[pl.BlockSpec(memory_space=pltpu.VMEM)],
        out_specs=pl.BlockSpec(memory_space=pltpu.VMEM),
        scratch_shapes=[
            pltpu.SemaphoreType.DMA(()),
            pltpu.SemaphoreType.DMA(()),
        ],
    )(w_own)


def _moe_body(x_ref, w1_ref, w2_ref, wg_ref, out_ref):
    e = pl.program_id(0)
    h = jnp.dot(
        x_ref[...],
        w1_ref[0].astype(jnp.bfloat16),
        preferred_element_type=jnp.float32,
    )
    h = jnp.maximum(h, 0.0).astype(jnp.bfloat16)
    contrib = jnp.dot(
        h, w2_ref[0].astype(jnp.bfloat16), preferred_element_type=jnp.float32
    )
    wg = wg_ref[...]
    col_ids = lax.broadcasted_iota(jnp.int32, wg.shape, 1)
    col = jnp.sum(jnp.where(col_ids == e, wg, 0.0), axis=1, keepdims=True)
    contrib = contrib * col

    @pl.when(e == 0)
    def _():
        out_ref[...] = contrib

    @pl.when(e != 0)
    def _():
        out_ref[...] += contrib


def _moe_compute(x_full, W1, W2, w_loc):
    t, d = x_full.shape
    e_loc, _, f = W1.shape
    return pl.pallas_call(
        _moe_body,
        grid=(e_loc,),
        in_specs=[
            pl.BlockSpec((t, d), lambda e: (0, 0)),
            pl.BlockSpec((1, d, f), lambda e: (e, 0, 0)),
            pl.BlockSpec((1, f, d), lambda e: (e, 0, 0)),
            pl.BlockSpec((t, e_loc), lambda e: (0, 0)),
        ],
        out_specs=pl.BlockSpec((t, d), lambda e: (0, 0)),
        out_shape=jax.ShapeDtypeStruct((t, d), jnp.float32),
        compiler_params=pltpu.CompilerParams(
            dimension_semantics=("arbitrary",),
            vmem_limit_bytes=100 * 1024 * 1024,
        ),
    )(x_full, W1, W2, w_loc)


def _combine_body(p_ref, out_ref, send_buf, recv_buf, send_sem, recv_sem):
    px = lax.axis_index("x")
    py = lax.axis_index("y")
    peer_x = 1 - px
    t_half = out_ref.shape[0]

    send_buf[...] = p_ref[pl.ds(peer_x * t_half, t_half), :].astype(
        jnp.bfloat16
    )
    rdma = pltpu.make_async_remote_copy(
        src_ref=send_buf,
        dst_ref=recv_buf,
        send_sem=send_sem,
        recv_sem=recv_sem,
        device_id=(peer_x, py),
        device_id_type=pl.DeviceIdType.MESH,
    )
    rdma.start()
    rdma.wait()
    out_ref[...] = p_ref[pl.ds(px * t_half, t_half), :] + recv_buf[
        ...
    ].astype(jnp.float32)


def _combine(partial, t_half):
    t, d = partial.shape
    return pl.pallas_call(
        _combine_body,
        out_shape=jax.ShapeDtypeStruct((t_half, d), jnp.float32),
        in_specs=[pl.BlockSpec(memory_space=pltpu.VMEM)],
        out_specs=pl.BlockSpec(memory_space=pltpu.VMEM),
        scratch_shapes=[
            pltpu.VMEM((t_half, d), jnp.bfloat16),
            pltpu.VMEM((t_half, d), jnp.bfloat16),
            pltpu.SemaphoreType.DMA(()),
            pltpu.SemaphoreType.DMA(()),
        ],
    )(partial)


def kernel(x, router, W1, W2):
    t_half, d = x.shape
    e_loc = W1.shape[0]
    px = lax.axis_index("x")

    xg, rg = _ag_xr(x.astype(jnp.bfloat16), router)
    x_full = xg.reshape(2 * t_half, d)
    router_full = jnp.concatenate([rg[0], rg[1]], axis=1)

    gates = jnp.dot(x, router_full, precision=lax.Precision.HIGHEST)
    top_v, top_i = lax.top_k(gates, TOPK)
    wts = jax.nn.softmax(top_v, axis=-1)
    w_own = (
        jnp.zeros((t_half, N_EXP), jnp.float32)
        .at[jnp.arange(t_half)[:, None], top_i]
        .add(wts)
    )

    wg = _ag_w(w_own)
    w_full = wg.reshape(2 * t_half, N_EXP)
    w_loc = lax.dynamic_slice_in_dim(w_full, px * e_loc, e_loc, axis=1)

    partial = _moe_compute(x_full, W1, W2, w_loc)

    return _combine(partial, t_half)


# device time: 108447 ns/iter; 2.8805x vs baseline; 2.8805x over previous
import jax
import jax.numpy as jnp
from jax import lax
from jax.experimental import pallas as pl
from jax.experimental.pallas import tpu as pltpu

N_EXP = 8
TOPK = 2


def _ag_xr_body(x_ref, r_ref, ox_ref, or_ref, send_sems, recv_sems):
    px = lax.axis_index("x")
    py = lax.axis_index("y")
    peer = (1 - px, py)

    ox_ref[pl.ds(px, 1)] = x_ref[...][None]
    or_ref[pl.ds(px, 1)] = r_ref[...][None]

    rdma_x = pltpu.make_async_remote_copy(
        src_ref=x_ref,
        dst_ref=ox_ref.at[px],
        send_sem=send_sems.at[0],
        recv_sem=recv_sems.at[0],
        device_id=peer,
        device_id_type=pl.DeviceIdType.MESH,
    )
    rdma_r = pltpu.make_async_remote_copy(
        src_ref=r_ref,
        dst_ref=or_ref.at[px],
        send_sem=send_sems.at[1],
        recv_sem=recv_sems.at[1],
        device_id=peer,
        device_id_type=pl.DeviceIdType.MESH,
    )
    rdma_x.start()
    rdma_r.start()
    rdma_x.wait()
    rdma_r.wait()


def _ag_xr(x_bf, r_shard):
    t_half, d = x_bf.shape
    t, e_loc = r_shard.shape
    return pl.pallas_call(
        _ag_xr_body,
        out_shape=(
            jax.ShapeDtypeStruct((2, t_half, d), jnp.bfloat16),
            jax.ShapeDtypeStruct((2, t, e_loc), jnp.float32),
        ),
        in_specs=[
            pl.BlockSpec(memory_space=pltpu.VMEM),
            pl.BlockSpec(memory_space=pltpu.VMEM),
        ],
        out_specs=(
            pl.BlockSpec(memory_space=pltpu.VMEM),
            pl.BlockSpec(memory_space=pltpu.VMEM),
        ),
        scratch_shapes=[
            pltpu.SemaphoreType.DMA((2,)),
            pltpu.SemaphoreType.DMA((2,)),
        ],
    )(x_bf, r_shard)


def _ag_w_body(w_ref, ow_ref, send_sem, recv_sem):
    px = lax.axis_index("x")
    py = lax.axis_index("y")
    ow_ref[pl.ds(px, 1)] = w_ref[...][None]
    rdma = pltpu.make_async_remote_copy(
        src_ref=w_ref,
        dst_ref=ow_ref.at[px],
        send_sem=send_sem,
        recv_sem=recv_sem,
        device_id=(1 - px, py),
        device_id_type=pl.DeviceIdType.MESH,
    )
    rdma.start()
    rdma.wait()


def _ag_w(w_own):
    t_half, ne = w_own.shape
    return pl.pallas_call(
        _ag_w_body,
        out_shape=jax.ShapeDtypeStruct((2, t_half, ne), jnp.float32),
        in_specs=[pl.BlockSpec(memory_space=pltpu.VMEM)],
        out_specs=pl.BlockSpec(memory_space=pltpu.VMEM),
        scratch_shapes=[
            pltpu.SemaphoreType.DMA(()),
            pltpu.SemaphoreType.DMA(()),
        ],
    )(w_own)


def _moe_body(x_ref, w1_ref, w2_ref, wg_ref, out_ref):
    e = pl.program_id(0)
    h = jnp.dot(
        x_ref[...],
        w1_ref[0].astype(jnp.bfloat16),
        preferred_element_type=jnp.float32,
    )
    h = jnp.maximum(h, 0.0).astype(jnp.bfloat16)
    contrib = jnp.dot(
        h, w2_ref[0].astype(jnp.bfloat16), preferred_element_type=jnp.float32
    )
    wg = wg_ref[...]
    col_ids = lax.broadcasted_iota(jnp.int32, wg.shape, 1)
    col = jnp.sum(jnp.where(col_ids == e, wg, 0.0), axis=1, keepdims=True)
    contrib = contrib * col

    @pl.when(e == 0)
    def _():
        out_ref[...] = contrib

    @pl.when(e != 0)
    def _():
        out_ref[...] += contrib


def _moe_compute(x_full, W1, W2, w_loc):
    t, d = x_full.shape
    e_loc, _, f = W1.shape
    return pl.pallas_call(
        _moe_body,
        grid=(e_loc,),
        in_specs=[
            pl.BlockSpec((t, d), lambda e: (0, 0)),
            pl.BlockSpec((1, d, f), lambda e: (e, 0, 0)),
            pl.BlockSpec((1, f, d), lambda e: (e, 0, 0)),
            pl.BlockSpec((t, e_loc), lambda e: (0, 0)),
        ],
        out_specs=pl.BlockSpec((t, d), lambda e: (0, 0)),
        out_shape=jax.ShapeDtypeStruct((t, d), jnp.float32),
        compiler_params=pltpu.CompilerParams(
            dimension_semantics=("arbitrary",),
            vmem_limit_bytes=100 * 1024 * 1024,
        ),
    )(x_full, W1, W2, w_loc)


def _combine_body(p_ref, out_ref, send_buf, recv_buf, send_sem, recv_sem):
    px = lax.axis_index("x")
    py = lax.axis_index("y")
    peer_x = 1 - px
    t_half = out_ref.shape[0]

    send_buf[...] = p_ref[pl.ds(peer_x * t_half, t_half), :].astype(
        jnp.bfloat16
    )
    rdma = pltpu.make_async_remote_copy(
        src_ref=send_buf,
        dst_ref=recv_buf,
        send_sem=send_sem,
        recv_sem=recv_sem,
        device_id=(peer_x, py),
        device_id_type=pl.DeviceIdType.MESH,
    )
    rdma.start()
    rdma.wait()
    out_ref[...] = p_ref[pl.ds(px * t_half, t_half), :] + recv_buf[
        ...
    ].astype(jnp.float32)


def _combine(partial, t_half):
    t, d = partial.shape
    return pl.pallas_call(
        _combine_body,
        out_shape=jax.ShapeDtypeStruct((t_half, d), jnp.float32),
        in_specs=[pl.BlockSpec(memory_space=pltpu.VMEM)],
        out_specs=pl.BlockSpec(memory_space=pltpu.VMEM),
        scratch_shapes=[
            pltpu.VMEM((t_half, d), jnp.bfloat16),
            pltpu.VMEM((t_half, d), jnp.bfloat16),
            pltpu.SemaphoreType.DMA(()),
            pltpu.SemaphoreType.DMA(()),
        ],
    )(partial)


def kernel(x, router, W1, W2):
    t_half, d = x.shape
    e_loc = W1.shape[0]
    px = lax.axis_index("x")

    xg, rg = _ag_xr(x.astype(jnp.bfloat16), router)
    x_full = xg.reshape(2 * t_half, d)
    router_full = jnp.concatenate([rg[0], rg[1]], axis=1)

    gates = jnp.dot(x, router_full, precision=lax.Precision.HIGHEST)
    top_v, top_i = lax.top_k(gates, TOPK)
    wts = jax.nn.softmax(top_v, axis=-1)
    one_hot = top_i[:, :, None] == jnp.arange(N_EXP)[None, None, :]
    w_own = jnp.sum(
        jnp.where(one_hot, wts[:, :, None], 0.0), axis=1
    )

    wg = _ag_w(w_own)
    w_full = wg.reshape(2 * t_half, N_EXP)
    w_loc = lax.dynamic_slice_in_dim(w_full, px * e_loc, e_loc, axis=1)

    partial = _moe_compute(x_full, W1, W2, w_loc)

    return _combine(partial, t_half)
